# baseline (speedup 1.0000x reference)
import functools

import jax
import jax.numpy as jnp
from jax import lax
from jax.experimental import pallas as pl
from jax.experimental.pallas import tpu as pltpu
from jax.experimental.pallas import tpu_sc as plsc

_W = 4
_K = 12
_NC = 2
_NS = 16
_L = 16


def _sort(v, descending=False):
    r = plsc.sort_key_val(v, v, descending=descending)
    if isinstance(r, (tuple, list)):
        r = r[0]
    return r


def _sc_pool_topk(groups, f, t):
    rows = groups * f
    nw = _NC * _NS
    rows_pw = rows // nw
    groups_pw = groups // nw
    ch = 64
    nvr = t // _L
    tw = t - _W + 1
    last_valid = tw - (nvr - 1) * _L
    scale = 1.0 / (_W * _K * f)
    neg = jnp.float32(-jnp.inf)

    mesh = plsc.VectorSubcoreMesh(
        core_axis_name="c", subcore_axis_name="s",
        num_cores=_NC, num_subcores=_NS,
    )

    @functools.partial(
        pl.kernel,
        mesh=mesh,
        out_type=jax.ShapeDtypeStruct((groups,), jnp.float32),
        compiler_params=pltpu.CompilerParams(needs_layout_passes=False),
        scratch_types=[
            pltpu.VMEM((ch * t + _L,), jnp.float32),
            pltpu.VMEM((ch * t + _L,), jnp.float32),
            pltpu.VMEM((groups_pw,), jnp.float32),
            pltpu.SemaphoreType.DMA,
            pltpu.SemaphoreType.DMA,
        ],
    )
    def run(x_hbm, out_hbm, xbuf0, xbuf1, resbuf, sem0, sem1):
        wid = lax.axis_index("s") * _NC + lax.axis_index("c")
        row0 = wid * rows_pw
        lane = lax.iota(jnp.int32, _L)
        topmask = lane >= (_L - _K)
        lastmask = lane < last_valid

        def start_chunk(chunk_idx, buf, sem):
            src = x_hbm.at[pl.ds((row0 + chunk_idx * ch) * t, ch * t)]
            pltpu.make_async_copy(src, buf.at[pl.ds(0, ch * t)], sem).start()

        start_chunk(0, xbuf0, sem0)
        start_chunk(1, xbuf1, sem1)

        ilv = 4

        def _wsum(buf, off):
            a0 = buf[pl.ds(off, _L)]
            a1 = buf[pl.ds(off + 1, _L)]
            a2 = buf[pl.ds(off + 2, _L)]
            a3 = buf[pl.ds(off + 3, _L)]
            return (a0 + a1) + (a2 + a3)

        def chunk_sum(buf, gacc):
            def blk(q, acc):
                r0 = q * ilv
                Rs = [jnp.full((_L,), neg) for _ in range(ilv)]
                for i in range(nvr):
                    ioff = i * _L
                    for j in range(ilv):
                        ws = _wsum(buf, (r0 + j) * t + ioff)
                        if i == nvr - 1:
                            ws = jnp.where(lastmask, ws, neg)
                        c_dsc = _sort(ws, descending=True)
                        Rs[j] = _sort(jnp.maximum(Rs[j], c_dsc))
                for j in range(ilv):
                    acc = acc + jnp.where(topmask, Rs[j], 0.0)
                return acc

            return lax.fori_loop(0, ch // ilv, blk, gacc)

        def group_body(g, _):
            pltpu.make_async_copy(
                x_hbm.at[pl.ds(0, ch * t)], xbuf0.at[pl.ds(0, ch * t)], sem0
            ).wait()
            gacc = chunk_sum(xbuf0, jnp.zeros((_L,), jnp.float32))

            @pl.when(g + 1 < groups_pw)
            def _():
                start_chunk(2 * (g + 1), xbuf0, sem0)

            pltpu.make_async_copy(
                x_hbm.at[pl.ds(0, ch * t)], xbuf1.at[pl.ds(0, ch * t)], sem1
            ).wait()
            gacc = chunk_sum(xbuf1, gacc)

            @pl.when(g + 1 < groups_pw)
            def _():
                start_chunk(2 * (g + 1) + 1, xbuf1, sem1)

            tot = plsc.cumsum(gacc) * scale
            plsc.store_scatter(
                resbuf, [jnp.full((_L,), g, jnp.int32)], tot,
                mask=lane == (_L - 1),
            )
            return 0

        lax.fori_loop(0, groups_pw, group_body, 0)
        pltpu.sync_copy(resbuf, out_hbm.at[pl.ds(wid * groups_pw, groups_pw)])

    return run


def kernel(x):
    b, c, f, t = x.shape
    if _W <= 1 or t < _W:
        return x.mean(axis=(-1, -2))
    xr = x.reshape(b * c * f * t)
    out = _sc_pool_topk(b * c, f, t)(xr)
    return out.reshape(b, c)


# speedup vs baseline: 1.9035x; 1.9035x over previous
import functools

import jax
import jax.numpy as jnp
from jax import lax
from jax.experimental import pallas as pl
from jax.experimental.pallas import tpu as pltpu
from jax.experimental.pallas import tpu_sc as plsc

_W = 4
_K = 12
_NC = 2
_NS = 16
_L = 16


def _sort(v, descending=False):
    r = plsc.sort_key_val(v, v, descending=descending)
    if isinstance(r, (tuple, list)):
        r = r[0]
    return r


def _sc_pool_topk(groups, f, t):
    rows = groups * f
    nw = _NC * _NS
    rows_pw = rows // nw
    groups_pw = groups // nw
    ch = 64
    nvr = t // _L
    tw = t - _W + 1
    last_valid = tw - (nvr - 1) * _L
    scale = 1.0 / (_W * _K * f)
    neg = jnp.float32(-jnp.inf)

    mesh = plsc.VectorSubcoreMesh(
        core_axis_name="c", subcore_axis_name="s",
        num_cores=_NC, num_subcores=_NS,
    )

    @functools.partial(
        pl.kernel,
        mesh=mesh,
        out_type=jax.ShapeDtypeStruct((groups,), jnp.float32),
        compiler_params=pltpu.CompilerParams(needs_layout_passes=False),
        scratch_types=[
            pltpu.VMEM((ch * t + _L,), jnp.float32),
            pltpu.VMEM((ch * t + _L,), jnp.float32),
            pltpu.VMEM((groups_pw,), jnp.float32),
            pltpu.SemaphoreType.DMA,
            pltpu.SemaphoreType.DMA,
        ],
    )
    def run(x_hbm, out_hbm, xbuf0, xbuf1, resbuf, sem0, sem1):
        wid = lax.axis_index("s") * _NC + lax.axis_index("c")
        row0 = wid * rows_pw
        lane = lax.iota(jnp.int32, _L)
        topmask = lane >= (_L - _K)
        lastmask = lane < last_valid

        def start_chunk(chunk_idx, buf, sem):
            src = x_hbm.at[pl.ds((row0 + chunk_idx * ch) * t, ch * t)]
            pltpu.make_async_copy(src, buf.at[pl.ds(0, ch * t)], sem).start()

        start_chunk(0, xbuf0, sem0)
        start_chunk(1, xbuf1, sem1)

        ilv = 8

        def _wsum(buf, off):
            a0 = buf[pl.ds(off, _L)]
            a1 = buf[pl.ds(off + 1, _L)]
            a2 = buf[pl.ds(off + 2, _L)]
            a3 = buf[pl.ds(off + 3, _L)]
            return (a0 + a1) + (a2 + a3)

        def chunk_sum(buf, gacc):
            def blk(q, acc):
                r0 = q * ilv

                @plsc.parallel_loop(
                    0, nvr - 1, unroll=4,
                    carry=tuple(jnp.full((_L,), neg) for _ in range(ilv)),
                )
                def Rs(i, Rs_in):
                    ioff = i * _L
                    out = []
                    for j in range(ilv):
                        ws = _wsum(buf, (r0 + j) * t + ioff)
                        c_dsc = _sort(ws, descending=True)
                        out.append(_sort(jnp.maximum(Rs_in[j], c_dsc)))
                    return tuple(out)
                ioff = (nvr - 1) * _L
                for j in range(ilv):
                    ws = jnp.where(
                        lastmask, _wsum(buf, (r0 + j) * t + ioff), neg
                    )
                    c_dsc = _sort(ws, descending=True)
                    R = _sort(jnp.maximum(Rs[j], c_dsc))
                    acc = acc + jnp.where(topmask, R, 0.0)
                return acc

            return lax.fori_loop(0, ch // ilv, blk, gacc)

        def group_body(g, _):
            pltpu.make_async_copy(
                x_hbm.at[pl.ds(0, ch * t)], xbuf0.at[pl.ds(0, ch * t)], sem0
            ).wait()
            gacc = chunk_sum(xbuf0, jnp.zeros((_L,), jnp.float32))

            @pl.when(g + 1 < groups_pw)
            def _():
                start_chunk(2 * (g + 1), xbuf0, sem0)

            pltpu.make_async_copy(
                x_hbm.at[pl.ds(0, ch * t)], xbuf1.at[pl.ds(0, ch * t)], sem1
            ).wait()
            gacc = chunk_sum(xbuf1, gacc)

            @pl.when(g + 1 < groups_pw)
            def _():
                start_chunk(2 * (g + 1) + 1, xbuf1, sem1)

            tot = plsc.cumsum(gacc) * scale
            plsc.store_scatter(
                resbuf, [jnp.full((_L,), g, jnp.int32)], tot,
                mask=lane == (_L - 1),
            )
            return 0

        lax.fori_loop(0, groups_pw, group_body, 0)
        pltpu.sync_copy(resbuf, out_hbm.at[pl.ds(wid * groups_pw, groups_pw)])

    return run


def kernel(x):
    b, c, f, t = x.shape
    if _W <= 1 or t < _W:
        return x.mean(axis=(-1, -2))
    xr = x.reshape(b * c * f * t)
    out = _sc_pool_topk(b * c, f, t)(xr)
    return out.reshape(b, c)


# speedup vs baseline: 2.7827x; 1.4619x over previous
import functools

import jax
import jax.numpy as jnp
from jax import lax
from jax.experimental import pallas as pl
from jax.experimental.pallas import tpu as pltpu
from jax.experimental.pallas import tpu_sc as plsc

_W = 4
_K = 12
_NC = 2
_NS = 16
_L = 16


def _sort(v, descending=False):
    r = plsc.sort_key_val(v, v, descending=descending)
    if isinstance(r, (tuple, list)):
        r = r[0]
    return r


def _sc_pool_topk(groups, f, t):
    rows = groups * f
    nw = _NC * _NS
    rows_pw = rows // nw
    groups_pw = groups // nw
    ch = 64
    nvr = t // _L
    tw = t - _W + 1
    last_valid = tw - (nvr - 1) * _L
    scale = 1.0 / (_W * _K * f)
    neg = jnp.float32(-jnp.inf)

    mesh = plsc.VectorSubcoreMesh(
        core_axis_name="c", subcore_axis_name="s",
        num_cores=_NC, num_subcores=_NS,
    )

    @functools.partial(
        pl.kernel,
        mesh=mesh,
        out_type=jax.ShapeDtypeStruct((groups,), jnp.float32),
        compiler_params=pltpu.CompilerParams(needs_layout_passes=False),
        scratch_types=[
            pltpu.VMEM((ch * t + _L,), jnp.float32),
            pltpu.VMEM((ch * t + _L,), jnp.float32),
            pltpu.VMEM((groups_pw,), jnp.float32),
            pltpu.SemaphoreType.DMA,
            pltpu.SemaphoreType.DMA,
        ],
    )
    def run(x_hbm, out_hbm, xbuf0, xbuf1, resbuf, sem0, sem1):
        wid = lax.axis_index("s") * _NC + lax.axis_index("c")
        row0 = wid * rows_pw
        lane = lax.iota(jnp.int32, _L)
        topmask = lane >= (_L - _K)
        lastmask = lane < last_valid

        def start_chunk(chunk_idx, buf, sem):
            src = x_hbm.at[pl.ds((row0 + chunk_idx * ch) * t, ch * t)]
            pltpu.make_async_copy(src, buf.at[pl.ds(0, ch * t)], sem).start()

        start_chunk(0, xbuf0, sem0)
        start_chunk(1, xbuf1, sem1)

        ilv = 8

        def _wsum(buf, off):
            a0 = buf[pl.ds(off, _L)]
            a1 = buf[pl.ds(off + 1, _L)]
            a2 = buf[pl.ds(off + 2, _L)]
            a3 = buf[pl.ds(off + 3, _L)]
            return (a0 + a1) + (a2 + a3)

        def chunk_sum(buf, gacc):
            def blk(q, acc):
                r0 = q * ilv

                @plsc.parallel_loop(
                    0, nvr - 1, unroll=1,
                    carry=tuple(jnp.full((_L,), neg) for _ in range(ilv)),
                )
                def Rs(i, Rs_in):
                    ioff = i * _L
                    out = []
                    for j in range(ilv):
                        ws = _wsum(buf, (r0 + j) * t + ioff)
                        c_dsc = _sort(ws, descending=True)
                        out.append(_sort(jnp.maximum(Rs_in[j], c_dsc)))
                    return tuple(out)
                ioff = (nvr - 1) * _L
                for j in range(ilv):
                    ws = jnp.where(
                        lastmask, _wsum(buf, (r0 + j) * t + ioff), neg
                    )
                    c_dsc = _sort(ws, descending=True)
                    R = _sort(jnp.maximum(Rs[j], c_dsc))
                    acc = acc + jnp.where(topmask, R, 0.0)
                return acc

            return lax.fori_loop(0, ch // ilv, blk, gacc)

        def group_body(g, _):
            pltpu.make_async_copy(
                x_hbm.at[pl.ds(0, ch * t)], xbuf0.at[pl.ds(0, ch * t)], sem0
            ).wait()
            gacc = chunk_sum(xbuf0, jnp.zeros((_L,), jnp.float32))

            @pl.when(g + 1 < groups_pw)
            def _():
                start_chunk(2 * (g + 1), xbuf0, sem0)

            pltpu.make_async_copy(
                x_hbm.at[pl.ds(0, ch * t)], xbuf1.at[pl.ds(0, ch * t)], sem1
            ).wait()
            gacc = chunk_sum(xbuf1, gacc)

            @pl.when(g + 1 < groups_pw)
            def _():
                start_chunk(2 * (g + 1) + 1, xbuf1, sem1)

            tot = plsc.cumsum(gacc) * scale
            plsc.store_scatter(
                resbuf, [jnp.full((_L,), g, jnp.int32)], tot,
                mask=lane == (_L - 1),
            )
            return 0

        lax.fori_loop(0, groups_pw, group_body, 0)
        pltpu.sync_copy(resbuf, out_hbm.at[pl.ds(wid * groups_pw, groups_pw)])

    return run


def kernel(x):
    b, c, f, t = x.shape
    if _W <= 1 or t < _W:
        return x.mean(axis=(-1, -2))
    xr = x.reshape(b * c * f * t)
    out = _sc_pool_topk(b * c, f, t)(xr)
    return out.reshape(b, c)
